# baseline (speedup 1.0000x reference)
import functools

import jax
import jax.numpy as jnp
import numpy as np
from jax import lax
from jax.experimental import pallas as pl
from jax.experimental.pallas import tpu as pltpu

NN = 50000
NE = 800000
NS = 400000
NH = 100
NT = 5
FO = NH // NT
NLAYERS = 2
_DEG_HIST = np.array([0.0, 1000.0, 5000.0, 10000.0, 15000.0, 10000.0, 5000.0, 3000.0, 1000.0])
_b = np.arange(_DEG_HIST.shape[0]).astype(np.float64)
ADL = float((np.log(_b + 1.0) * _DEG_HIST).sum() / _DEG_HIST.sum())

_RB = 400


def _bn_stats_body(x_ref, s_ref, q_ref, acc_s, acc_q):
    i = pl.program_id(0)

    @pl.when(i == 0)
    def _init():
        acc_s[...] = jnp.zeros_like(acc_s)
        acc_q[...] = jnp.zeros_like(acc_q)

    xb = x_ref[...]
    acc_s[...] += jnp.sum(xb, axis=0, keepdims=True)
    acc_q[...] += jnp.sum(xb * xb, axis=0, keepdims=True)

    @pl.when(i == pl.num_programs(0) - 1)
    def _fin():
        s_ref[...] = acc_s[...]
        q_ref[...] = acc_q[...]


def _bn_apply_body(x_ref, c_ref, s_ref, q_ref, g_ref, b_ref, o_ref):
    m = s_ref[...] / NN
    v = q_ref[...] / NN - m * m
    inv = jax.lax.rsqrt(v + 1e-5)
    c = c_ref[...]
    h = g_ref[...] * (c - m) * inv + b_ref[...]
    o_ref[...] = (x_ref[...] + jnp.maximum(h, 0.0)) * 0.5


def _bn_residual(x, conv, gamma, beta):
    nb = NN // _RB
    s, q = pl.pallas_call(
        _bn_stats_body,
        grid=(nb,),
        in_specs=[pl.BlockSpec((_RB, NH), lambda i: (i, 0))],
        out_specs=[
            pl.BlockSpec((1, NH), lambda i: (0, 0)),
            pl.BlockSpec((1, NH), lambda i: (0, 0)),
        ],
        out_shape=[
            jax.ShapeDtypeStruct((1, NH), jnp.float32),
            jax.ShapeDtypeStruct((1, NH), jnp.float32),
        ],
        scratch_shapes=[
            pltpu.VMEM((1, NH), jnp.float32),
            pltpu.VMEM((1, NH), jnp.float32),
        ],
    )(conv)
    out = pl.pallas_call(
        _bn_apply_body,
        grid=(nb,),
        in_specs=[
            pl.BlockSpec((_RB, NH), lambda i: (i, 0)),
            pl.BlockSpec((_RB, NH), lambda i: (i, 0)),
            pl.BlockSpec((1, NH), lambda i: (0, 0)),
            pl.BlockSpec((1, NH), lambda i: (0, 0)),
            pl.BlockSpec((1, NH), lambda i: (0, 0)),
            pl.BlockSpec((1, NH), lambda i: (0, 0)),
        ],
        out_specs=pl.BlockSpec((_RB, NH), lambda i: (i, 0)),
        out_shape=jax.ShapeDtypeStruct((NN, NH), jnp.float32),
    )(x, conv, s, q, gamma.reshape(1, NH), beta.reshape(1, NH))
    return out


def _edge_graph(edge_index, simp):
    flags = jnp.concatenate(
        [jnp.zeros((1,), jnp.int32), (simp[1:] != simp[:-1]).astype(jnp.int32)]
    )
    inv = jnp.cumsum(flags)
    cnt = jax.ops.segment_sum(
        jnp.ones((NE,), jnp.float32), inv, NS, indices_are_sorted=True
    )
    valid = cnt > 0
    cnt_c = jnp.maximum(cnt, 1.0)
    ei_sum = jax.ops.segment_sum(edge_index.T, inv, NS, indices_are_sorted=True)
    nei = (ei_sum // cnt_c.astype(ei_sum.dtype)[:, None]).T
    src_new = nei[0]
    dst_new = jnp.where(valid, nei[1], jnp.int32(NN))
    perm = jnp.argsort(dst_new)
    ds_sorted = dst_new[perm]
    return inv, src_new, perm, ds_sorted


_SB = 256
_SNB = (NS + _SB - 1) // _SB
_SNSP = _SNB * _SB
_SBO = 512
_SNOB = 99
_SROWS = _SBO * _SNOB
_SC_COLS = 2 * NH + 1


def _sums_body(ds0_ref, offf_ref, us_ref, out_ref, acc_ref):
    i = pl.program_id(0)

    @pl.when(i == 0)
    def _init():
        acc_ref[...] = jnp.zeros_like(acc_ref)

    @pl.when(i < _SNB)
    def _accum():
        usb = us_ref[0]
        offc = offf_ref[0]
        ds0 = pl.multiple_of(ds0_ref[i], 8)
        data = jnp.concatenate(
            [usb, usb * usb, jnp.ones((_SB, 1), jnp.float32)], axis=1)
        ioc = lax.broadcasted_iota(jnp.int32, (_SB, _SB), 1).astype(jnp.float32)
        m = (offc == ioc)
        m = m.astype(jnp.float32)
        partial = jax.lax.dot_general(
            m, data, (((0,), (0,)), ((), ())),
            preferred_element_type=jnp.float32)
        acc_ref[pl.ds(ds0, _SB), :] += partial
        nbig = jnp.sum((offc >= _SB).astype(jnp.float32))

        @pl.when(nbig > 0)
        def _rare():
            nw = (jnp.int32(_SROWS - _SB) - ds0) // _SB + 1

            def win(w, carry):
                mw = (offc - (w * _SB).astype(jnp.float32) == ioc)
                pw = jax.lax.dot_general(
                    mw.astype(jnp.float32), data, (((0,), (0,)), ((), ())),
                    preferred_element_type=jnp.float32)
                base = pl.multiple_of(ds0 + w * _SB, 8)
                acc_ref[pl.ds(base, _SB), :] += pw
                return carry

            lax.fori_loop(1, nw, win, 0)

    @pl.when(i >= _SNB)
    def _emit():
        jb = pl.multiple_of((i - _SNB) * _SBO, 8)
        out_ref[...] = acc_ref[pl.ds(jb, _SBO), :]


def _sorted_sums(ds_sorted, us):
    dsp = jnp.concatenate(
        [ds_sorted, jnp.full((_SNSP - NS,), NN, jnp.int32)])
    usp = jnp.pad(us, ((0, _SNSP - NS), (0, 0)))
    ds0s = (dsp[::_SB] // 8) * 8
    offs = dsp.reshape(_SNB, _SB) - ds0s[:, None]
    offf = offs.reshape(_SNB, _SB, 1).astype(jnp.float32)
    grid = (_SNB + _SNOB,)
    out = pl.pallas_call(
        _sums_body,
        grid_spec=pltpu.PrefetchScalarGridSpec(
            num_scalar_prefetch=1,
            grid=grid,
            in_specs=[
                pl.BlockSpec((1, _SB, 1),
                             lambda i, s: (jnp.minimum(i, _SNB - 1), 0, 0)),
                pl.BlockSpec((1, _SB, NH),
                             lambda i, s: (jnp.minimum(i, _SNB - 1), 0, 0)),
            ],
            out_specs=pl.BlockSpec(
                (_SBO, _SC_COLS),
                lambda i, s: (jnp.maximum(i - _SNB, 0), 0)),
            scratch_shapes=[pltpu.VMEM((_SROWS, _SC_COLS), jnp.float32)],
        ),
        out_shape=jax.ShapeDtypeStruct((_SROWS, _SC_COLS), jnp.float32),
    )(ds0s, offf, usp.reshape(_SNB, _SB, NH))
    return out


def _pna(p, x, src, perm, ds_sorted, nrest):
    wflat = jnp.transpose(p["Wpre"], (1, 0, 2)).reshape(3 * NH, NT * FO)
    bflat = p["bpre"].reshape(NT * FO)
    wd, ws, we2 = wflat[:NH], wflat[NH:2 * NH], wflat[2 * NH:]
    a = x @ wd
    u = (x @ ws)[src] + nrest @ (p["We"] @ we2) + (p["be"] @ we2 + bflat)
    us = u[perm]
    sums = _sorted_sums(ds_sorted, us)
    mnmx = jax.ops.segment_min(
        jnp.concatenate([us, -us], axis=1), ds_sorted, NN,
        indices_are_sorted=True)
    S, Q = sums[:NN, :NT * FO], sums[:NN, NT * FO:2 * NT * FO]
    MN, MX = mnmx[:, :NT * FO], -mnmx[:, NT * FO:]
    cnt = sums[:NN, 2 * NT * FO]
    cnt_c = jnp.maximum(cnt, 1.0)[:, None]
    has = (cnt > 0)[:, None]
    mean = jnp.where(has, a + S / cnt_c, 0.0)
    var = jnp.maximum(Q / cnt_c - (S / cnt_c) ** 2, 0.0)
    std = jnp.sqrt(var + 1e-5)
    mn = jnp.where(has, a + MN, 0.0)
    mx = jnp.where(has, a + MX, 0.0)
    r3 = lambda t: t.reshape(NN, NT, FO)
    aggr = jnp.concatenate([r3(mean), r3(mn), r3(mx), r3(std)], axis=-1)
    cnt_c1 = jnp.maximum(cnt, 1.0)
    amp = (jnp.log(cnt_c1 + 1.0) / ADL)[:, None, None]
    att = (ADL / jnp.log(cnt_c1 + 1.0))[:, None, None]
    scaled = jnp.concatenate([aggr, aggr * amp, aggr * att], axis=-1)
    xt = jnp.broadcast_to(x[:, None, :], (NN, NT, NH))
    out = jnp.concatenate([xt, scaled], axis=-1)
    outs = jnp.einsum("ntf,tfo->nto", out, p["Wpost"]) + p["bpost"]
    return outs.reshape(NN, NT * FO) @ p["Wlin"] + p["blin"]


def kernel(x, edge_index, edge_attr, simp_edge_batch, params):
    src = edge_index[0]
    inv, src_new, perm, ds_sorted = _edge_graph(edge_index, simp_edge_batch)
    for l in range(NLAYERS):
        p = params[l]
        rest = edge_attr[:, 1:]
        nrest = jax.ops.segment_sum(rest, inv, NS, indices_are_sorted=True)
        conv = _pna(p, x, src_new, perm, ds_sorted, nrest)
        x = _bn_residual(x, conv, p["gamma"], p["beta"])
        ts = edge_attr[:, :1]
        remapped = nrest[inv]
        h = jnp.concatenate([x[src], remapped, rest], axis=-1)
        h = jax.nn.relu(h @ p["Wm1"] + p["bm1"]) @ p["Wm2"] + p["bm2"]
        rest = rest + h * 0.5
        edge_attr = jnp.concatenate([ts, rest], axis=1)
    return x, edge_attr


# speedup vs baseline: 1.1793x; 1.1793x over previous
import functools

import jax
import jax.numpy as jnp
import numpy as np
from jax import lax
from jax.experimental import pallas as pl
from jax.experimental.pallas import tpu as pltpu

NN = 50000
NE = 800000
NS = 400000
NH = 100
NT = 5
FO = NH // NT
NLAYERS = 2
_DEG_HIST = np.array([0.0, 1000.0, 5000.0, 10000.0, 15000.0, 10000.0, 5000.0, 3000.0, 1000.0])
_b = np.arange(_DEG_HIST.shape[0]).astype(np.float64)
ADL = float((np.log(_b + 1.0) * _DEG_HIST).sum() / _DEG_HIST.sum())

_RB = 400


def _bn_stats_body(x_ref, s_ref, q_ref, acc_s, acc_q):
    i = pl.program_id(0)

    @pl.when(i == 0)
    def _init():
        acc_s[...] = jnp.zeros_like(acc_s)
        acc_q[...] = jnp.zeros_like(acc_q)

    xb = x_ref[...]
    acc_s[...] += jnp.sum(xb, axis=0, keepdims=True)
    acc_q[...] += jnp.sum(xb * xb, axis=0, keepdims=True)

    @pl.when(i == pl.num_programs(0) - 1)
    def _fin():
        s_ref[...] = acc_s[...]
        q_ref[...] = acc_q[...]


def _bn_apply_body(x_ref, c_ref, s_ref, q_ref, g_ref, b_ref, o_ref):
    m = s_ref[...] / NN
    v = q_ref[...] / NN - m * m
    inv = jax.lax.rsqrt(v + 1e-5)
    c = c_ref[...]
    h = g_ref[...] * (c - m) * inv + b_ref[...]
    o_ref[...] = (x_ref[...] + jnp.maximum(h, 0.0)) * 0.5


def _bn_residual(x, conv, gamma, beta):
    nb = NN // _RB
    s, q = pl.pallas_call(
        _bn_stats_body,
        grid=(nb,),
        in_specs=[pl.BlockSpec((_RB, NH), lambda i: (i, 0))],
        out_specs=[
            pl.BlockSpec((1, NH), lambda i: (0, 0)),
            pl.BlockSpec((1, NH), lambda i: (0, 0)),
        ],
        out_shape=[
            jax.ShapeDtypeStruct((1, NH), jnp.float32),
            jax.ShapeDtypeStruct((1, NH), jnp.float32),
        ],
        scratch_shapes=[
            pltpu.VMEM((1, NH), jnp.float32),
            pltpu.VMEM((1, NH), jnp.float32),
        ],
    )(conv)
    out = pl.pallas_call(
        _bn_apply_body,
        grid=(nb,),
        in_specs=[
            pl.BlockSpec((_RB, NH), lambda i: (i, 0)),
            pl.BlockSpec((_RB, NH), lambda i: (i, 0)),
            pl.BlockSpec((1, NH), lambda i: (0, 0)),
            pl.BlockSpec((1, NH), lambda i: (0, 0)),
            pl.BlockSpec((1, NH), lambda i: (0, 0)),
            pl.BlockSpec((1, NH), lambda i: (0, 0)),
        ],
        out_specs=pl.BlockSpec((_RB, NH), lambda i: (i, 0)),
        out_shape=jax.ShapeDtypeStruct((NN, NH), jnp.float32),
    )(x, conv, s, q, gamma.reshape(1, NH), beta.reshape(1, NH))
    return out


def _edge_graph(edge_index, simp):
    flags = jnp.concatenate(
        [jnp.zeros((1,), jnp.int32), (simp[1:] != simp[:-1]).astype(jnp.int32)]
    )
    inv = jnp.cumsum(flags)
    cnt = jax.ops.segment_sum(
        jnp.ones((NE,), jnp.float32), inv, NS, indices_are_sorted=True
    )
    valid = cnt > 0
    cnt_c = jnp.maximum(cnt, 1.0)
    ei_sum = jax.ops.segment_sum(edge_index.T, inv, NS, indices_are_sorted=True)
    nei = (ei_sum // cnt_c.astype(ei_sum.dtype)[:, None]).T
    src_new = nei[0]
    dst_new = jnp.where(valid, nei[1], jnp.int32(NN))
    perm = jnp.argsort(dst_new)
    ds_sorted = dst_new[perm]
    return inv, src_new, perm, ds_sorted


_SB = 256
_SNB = (NS + _SB - 1) // _SB
_SNSP = _SNB * _SB
_SBO = 512
_SNOB = 99
_SROWS = _SBO * _SNOB
_SC_COLS = 2 * NH + 1


def _sums_body(ds0_ref, offf_ref, us_ref, out_ref, acc_ref):
    i = pl.program_id(0)

    @pl.when(i == 0)
    def _init():
        acc_ref[...] = jnp.zeros_like(acc_ref)

    @pl.when(i < _SNB)
    def _accum():
        usb = us_ref[0]
        offc = offf_ref[0]
        ds0 = pl.multiple_of(ds0_ref[i], 8)
        data = jnp.concatenate(
            [usb, usb * usb, jnp.ones((_SB, 1), jnp.float32)], axis=1)
        ioc = lax.broadcasted_iota(jnp.int32, (_SB, _SB), 1).astype(jnp.float32)
        m = (offc == ioc)
        m = m.astype(jnp.float32)
        partial = jax.lax.dot_general(
            m, data, (((0,), (0,)), ((), ())),
            preferred_element_type=jnp.float32)
        acc_ref[pl.ds(ds0, _SB), :] += partial
        nbig = jnp.sum((offc >= _SB).astype(jnp.float32))

        @pl.when(nbig > 0)
        def _rare():
            nw = (jnp.int32(_SROWS - _SB) - ds0) // _SB + 1

            def win(w, carry):
                mw = (offc - (w * _SB).astype(jnp.float32) == ioc)
                pw = jax.lax.dot_general(
                    mw.astype(jnp.float32), data, (((0,), (0,)), ((), ())),
                    preferred_element_type=jnp.float32)
                base = pl.multiple_of(ds0 + w * _SB, 8)
                acc_ref[pl.ds(base, _SB), :] += pw
                return carry

            lax.fori_loop(1, nw, win, 0)

    @pl.when(i >= _SNB)
    def _emit():
        jb = pl.multiple_of((i - _SNB) * _SBO, 8)
        out_ref[...] = acc_ref[pl.ds(jb, _SBO), :]


def _sorted_sums(ds_sorted, us):
    dsp = jnp.concatenate(
        [ds_sorted, jnp.full((_SNSP - NS,), NN, jnp.int32)])
    usp = jnp.pad(us, ((0, _SNSP - NS), (0, 0)))
    ds0s = (dsp[::_SB] // 8) * 8
    offs = dsp.reshape(_SNB, _SB) - ds0s[:, None]
    offf = offs.reshape(_SNB, _SB, 1).astype(jnp.float32)
    grid = (_SNB + _SNOB,)
    out = pl.pallas_call(
        _sums_body,
        grid_spec=pltpu.PrefetchScalarGridSpec(
            num_scalar_prefetch=1,
            grid=grid,
            in_specs=[
                pl.BlockSpec((1, _SB, 1),
                             lambda i, s: (jnp.minimum(i, _SNB - 1), 0, 0)),
                pl.BlockSpec((1, _SB, NH),
                             lambda i, s: (jnp.minimum(i, _SNB - 1), 0, 0)),
            ],
            out_specs=pl.BlockSpec(
                (_SBO, _SC_COLS),
                lambda i, s: (jnp.maximum(i - _SNB, 0), 0)),
            scratch_shapes=[pltpu.VMEM((_SROWS, _SC_COLS), jnp.float32)],
        ),
        out_shape=jax.ShapeDtypeStruct((_SROWS, _SC_COLS), jnp.float32),
    )(ds0s, offf, usp.reshape(_SNB, _SB, NH))
    return out


def _pna(p, x, src, perm, ds_sorted, nrest):
    wflat = jnp.transpose(p["Wpre"], (1, 0, 2)).reshape(3 * NH, NT * FO)
    bflat = p["bpre"].reshape(NT * FO)
    wd, ws, we2 = wflat[:NH], wflat[NH:2 * NH], wflat[2 * NH:]
    a = x @ wd
    u = (x @ ws)[src] + nrest @ (p["We"] @ we2) + (p["be"] @ we2 + bflat)
    us = u[perm]
    sums = _sorted_sums(ds_sorted, us)
    mnmx = jax.ops.segment_min(
        jnp.concatenate([us, -us], axis=1), ds_sorted, NN,
        indices_are_sorted=True)
    S, Q = sums[:NN, :NT * FO], sums[:NN, NT * FO:2 * NT * FO]
    MN, MX = mnmx[:, :NT * FO], -mnmx[:, NT * FO:]
    cnt = sums[:NN, 2 * NT * FO]
    cnt_c = jnp.maximum(cnt, 1.0)[:, None]
    has = (cnt > 0)[:, None]
    mean = jnp.where(has, a + S / cnt_c, 0.0)
    var = jnp.maximum(Q / cnt_c - (S / cnt_c) ** 2, 0.0)
    std = jnp.sqrt(var + 1e-5)
    mn = jnp.where(has, a + MN, 0.0)
    mx = jnp.where(has, a + MX, 0.0)
    r3 = lambda t: t.reshape(NN, NT, FO)
    aggr = jnp.concatenate([r3(mean), r3(mn), r3(mx), r3(std)], axis=-1)
    cnt_c1 = jnp.maximum(cnt, 1.0)
    amp = (jnp.log(cnt_c1 + 1.0) / ADL)[:, None, None]
    att = (ADL / jnp.log(cnt_c1 + 1.0))[:, None, None]
    scaled = jnp.concatenate([aggr, aggr * amp, aggr * att], axis=-1)
    xt = jnp.broadcast_to(x[:, None, :], (NN, NT, NH))
    out = jnp.concatenate([xt, scaled], axis=-1)
    outs = jnp.einsum("ntf,tfo->nto", out, p["Wpost"]) + p["bpost"]
    return outs.reshape(NN, NT * FO) @ p["Wlin"] + p["blin"]


_MB = 800


def _mlp_body(xg_ref, rm_ref, rs_ref, w1a_ref, w1b_ref, w1c_ref, b1_ref,
              w2_ref, b2_ref, o_ref):
    t = (jnp.dot(xg_ref[...], w1a_ref[...], preferred_element_type=jnp.float32)
         + jnp.dot(rm_ref[...], w1b_ref[...], preferred_element_type=jnp.float32)
         + jnp.dot(rs_ref[...], w1c_ref[...], preferred_element_type=jnp.float32)
         + b1_ref[...])
    t = jnp.maximum(t, 0.0)
    o_ref[...] = rs_ref[...] + (
        jnp.dot(t, w2_ref[...], preferred_element_type=jnp.float32)
        + b2_ref[...]) * 0.5


def _edge_mlp(xg, rm, rest, p):
    nb = NE // _MB
    row = lambda i: (i, 0)
    full = lambda i: (0, 0)
    return pl.pallas_call(
        _mlp_body,
        grid=(nb,),
        in_specs=[
            pl.BlockSpec((_MB, NH), row),
            pl.BlockSpec((_MB, NH), row),
            pl.BlockSpec((_MB, NH), row),
            pl.BlockSpec((NH, NH), full),
            pl.BlockSpec((NH, NH), full),
            pl.BlockSpec((NH, NH), full),
            pl.BlockSpec((1, NH), full),
            pl.BlockSpec((NH, NH), full),
            pl.BlockSpec((1, NH), full),
        ],
        out_specs=pl.BlockSpec((_MB, NH), row),
        out_shape=jax.ShapeDtypeStruct((NE, NH), jnp.float32),
    )(xg, rm, rest,
      p["Wm1"][:NH], p["Wm1"][NH:2 * NH], p["Wm1"][2 * NH:],
      p["bm1"].reshape(1, NH), p["Wm2"], p["bm2"].reshape(1, NH))


def kernel(x, edge_index, edge_attr, simp_edge_batch, params):
    src = edge_index[0]
    inv, src_new, perm, ds_sorted = _edge_graph(edge_index, simp_edge_batch)
    for l in range(NLAYERS):
        p = params[l]
        rest = edge_attr[:, 1:]
        nrest = jax.ops.segment_sum(rest, inv, NS, indices_are_sorted=True)
        conv = _pna(p, x, src_new, perm, ds_sorted, nrest)
        x = _bn_residual(x, conv, p["gamma"], p["beta"])
        ts = edge_attr[:, :1]
        newrest = _edge_mlp(x[src], nrest[inv], rest, p)
        edge_attr = jnp.concatenate([ts, newrest], axis=1)
    return x, edge_attr
